# baseline (speedup 1.0000x reference)
import jax
import jax.numpy as jnp
from jax import lax
from jax.experimental import pallas as pl
from jax.experimental.pallas import tpu as pltpu
from jax.experimental.pallas import tpu_sc as plsc

N = 10000
E = 320000
L = 100000
D = 128
H = 256
O = 128
DEC_H = 64

NC = 2
NS = 16
NW = NC * NS

_MESH = plsc.VectorSubcoreMesh(core_axis_name="c", subcore_axis_name="s")
_SC_PARAMS = pltpu.CompilerParams(needs_layout_passes=False,
                                  use_tc_tiling_on_sc=False)

SEG_CHUNK = 120
SEG_NCHUNK = 84
ECP = SEG_CHUNK * SEG_NCHUNK
EP = NW * ECP
NP = 10240
ROWS_PER_TILE = NP // NS


def _make_seg_sum(with_counts: bool):
    out_type = [jax.ShapeDtypeStruct((NC, NP, D), jnp.float32)]
    if with_counts:
        out_type.append(jax.ShapeDtypeStruct((NC, NP), jnp.float32))
    scratch = [
        pltpu.VMEM_SHARED((NP, D), jnp.float32),
        [pltpu.VMEM((SEG_CHUNK,), jnp.int32) for _ in range(3)],
        [pltpu.VMEM((SEG_CHUNK,), jnp.int32) for _ in range(3)],
        [pltpu.VMEM((SEG_CHUNK, D), jnp.float32) for _ in range(3)],
        [pltpu.SemaphoreType.DMA for _ in range(3)],
        [pltpu.SemaphoreType.DMA for _ in range(3)],
        [pltpu.SemaphoreType.DMA for _ in range(3)],
    ]
    if with_counts:
        scratch += [
            pltpu.VMEM_SHARED((NP,), jnp.float32),
            pltpu.VMEM((SEG_CHUNK,), jnp.float32),
        ]

    def body(table, src, dst, zeros2d, zeros1d, ones, *rest):
        if with_counts:
            (agg_out, cnt_out, agg_sp, sidx, didx, rows,
             isem, gsem, ssem, cnt_sp, ones_v) = rest
        else:
            (agg_out, agg_sp, sidx, didx, rows, isem, gsem, ssem) = rest
        cid = lax.axis_index("c")
        sid = lax.axis_index("s")
        wid = sid * NC + cid
        rbase = pl.multiple_of(sid * ROWS_PER_TILE, 8)
        pltpu.sync_copy(zeros2d.at[pl.ds(rbase, ROWS_PER_TILE)],
                        agg_sp.at[pl.ds(rbase, ROWS_PER_TILE)])
        if with_counts:
            pltpu.sync_copy(zeros1d.at[pl.ds(rbase, ROWS_PER_TILE)],
                            cnt_sp.at[pl.ds(rbase, ROWS_PER_TILE)])
            pltpu.sync_copy(ones, ones_v)
        plsc.subcore_barrier()

        ebase = wid * ECP

        def issue_idx(c, j):
            base = pl.multiple_of(ebase + c * SEG_CHUNK, 8)
            pltpu.async_copy(src.at[pl.ds(base, SEG_CHUNK)], sidx[j], isem[j])
            pltpu.async_copy(dst.at[pl.ds(base, SEG_CHUNK)], didx[j], isem[j])

        def wait_idx(j):
            pltpu.make_async_copy(src.at[pl.ds(0, SEG_CHUNK)], sidx[j], isem[j]).wait()
            pltpu.make_async_copy(dst.at[pl.ds(0, SEG_CHUNK)], didx[j], isem[j]).wait()

        def issue_gather(j):
            pltpu.async_copy(table.at[sidx[j]], rows[j], gsem[j])

        def wait_gather(j):
            pltpu.make_async_copy(table.at[sidx[j]], rows[j], gsem[j]).wait()

        def issue_scatter(j):
            pltpu.async_copy(rows[j], agg_sp.at[didx[j]], ssem[j], add=True)
            if with_counts:
                pltpu.async_copy(ones_v, cnt_sp.at[didx[j]], ssem[j], add=True)

        def wait_scatter(j):
            pltpu.make_async_copy(rows[j], agg_sp.at[didx[j]], ssem[j]).wait()
            if with_counts:
                pltpu.make_async_copy(ones_v, cnt_sp.at[didx[j]], ssem[j]).wait()

        def triple(i, carry):
            for j3 in range(3):
                c = 3 * i + j3

                @pl.when(c >= 3)
                def _(j=j3):
                    wait_scatter(j)

                @pl.when(c <= SEG_NCHUNK - 1)
                def _(c=c, j=j3):
                    issue_idx(c, j)

                @pl.when((c >= 1) & (c <= SEG_NCHUNK))
                def _(j=(j3 + 2) % 3):
                    wait_idx(j)
                    issue_gather(j)

                @pl.when((c >= 2) & (c <= SEG_NCHUNK + 1))
                def _(j=(j3 + 1) % 3):
                    wait_gather(j)
                    issue_scatter(j)

            return carry

        lax.fori_loop(0, (SEG_NCHUNK + 3) // 3, triple, 0)

        plsc.subcore_barrier()
        pltpu.sync_copy(agg_sp.at[pl.ds(rbase, ROWS_PER_TILE)],
                        agg_out.at[cid, pl.ds(rbase, ROWS_PER_TILE)])
        if with_counts:
            pltpu.sync_copy(cnt_sp.at[pl.ds(rbase, ROWS_PER_TILE)],
                            cnt_out.at[cid, pl.ds(rbase, ROWS_PER_TILE)])

    return pl.kernel(body, out_type=tuple(out_type), mesh=_MESH,
                     compiler_params=_SC_PARAMS, scratch_types=scratch)


_seg_sum_counts = _make_seg_sum(True)
_seg_sum = _make_seg_sum(False)

BM = 1000


def _enc1_body(x_ref, p_ref, cnt_ref, w1lt_ref, w1rt_ref, b1l_ref, g_ref,
               b_ref, h0_ref, h1_ref, inv_ref):
    cnt = cnt_ref[:, 0] + cnt_ref[:, 1]
    inv = 1.0 / jnp.maximum(cnt, 1.0)
    mean = (p_ref[0] + p_ref[1]) * inv[:, None]
    pre = (jnp.dot(mean, w1lt_ref[...], preferred_element_type=jnp.float32)
           + jnp.dot(x_ref[...], w1rt_ref[...], preferred_element_type=jnp.float32)
           + b1l_ref[...])
    mu = jnp.mean(pre, axis=-1, keepdims=True)
    var = jnp.mean((pre - mu) ** 2, axis=-1, keepdims=True)
    hh = (pre - mu) * lax.rsqrt(var + 1e-5) * g_ref[...] + b_ref[...]
    hh = jnp.maximum(hh, 0.0)
    h0_ref[...] = hh[:, :O]
    h1_ref[...] = hh[:, O:]
    inv_ref[...] = inv[:, None]


def _encoder1(x, parts, cnt_parts, w1lt, w1rt, b1l, ln_g, ln_b):
    grid = (N // BM,)
    return pl.pallas_call(
        _enc1_body,
        grid=grid,
        in_specs=[
            pl.BlockSpec((BM, D), lambda i: (i, 0)),
            pl.BlockSpec((NC, BM, D), lambda i: (0, i, 0)),
            pl.BlockSpec((BM, NC), lambda i: (i, 0)),
            pl.BlockSpec((D, H), lambda i: (0, 0)),
            pl.BlockSpec((D, H), lambda i: (0, 0)),
            pl.BlockSpec((1, H), lambda i: (0, 0)),
            pl.BlockSpec((1, H), lambda i: (0, 0)),
            pl.BlockSpec((1, H), lambda i: (0, 0)),
        ],
        out_specs=[
            pl.BlockSpec((BM, O), lambda i: (i, 0)),
            pl.BlockSpec((BM, O), lambda i: (i, 0)),
            pl.BlockSpec((BM, 1), lambda i: (i, 0)),
        ],
        out_shape=[
            jax.ShapeDtypeStruct((N, O), jnp.float32),
            jax.ShapeDtypeStruct((N, O), jnp.float32),
            jax.ShapeDtypeStruct((N, 1), jnp.float32),
        ],
    )(x, parts, cnt_parts, w1lt, w1rt, b1l, ln_g, ln_b)


def _enc2_body(h0_ref, h1_ref, a0_ref, a1_ref, inv_ref, w2lt_ref, w2rt_ref,
               b2l_ref, dw1t_ref, bpq_ref, p_ref, q_ref):
    inv = inv_ref[...]
    m0 = (a0_ref[0] + a0_ref[1]) * inv
    m1 = (a1_ref[0] + a1_ref[1]) * inv
    w2lt = w2lt_ref[...]
    w2rt = w2rt_ref[...]
    z = (jnp.dot(m0, w2lt[:O], preferred_element_type=jnp.float32)
         + jnp.dot(m1, w2lt[O:], preferred_element_type=jnp.float32)
         + jnp.dot(h0_ref[...], w2rt[:O], preferred_element_type=jnp.float32)
         + jnp.dot(h1_ref[...], w2rt[O:], preferred_element_type=jnp.float32)
         + b2l_ref[...])
    nrm = jnp.maximum(jnp.sqrt(jnp.sum(z * z, axis=-1, keepdims=True)), 1e-12)
    z = z / nrm
    pq = jnp.dot(z, dw1t_ref[...], preferred_element_type=jnp.float32) + bpq_ref[...]
    p_ref[...] = pq[:, :DEC_H]
    q_ref[...] = pq[:, DEC_H:]


def _encoder2(h0, h1, a0, a1, inv_cnt, w2lt, w2rt, b2l, dw1t, bpq):
    grid = (N // BM,)
    return pl.pallas_call(
        _enc2_body,
        grid=grid,
        in_specs=[
            pl.BlockSpec((BM, O), lambda i: (i, 0)),
            pl.BlockSpec((BM, O), lambda i: (i, 0)),
            pl.BlockSpec((NC, BM, O), lambda i: (0, i, 0)),
            pl.BlockSpec((NC, BM, O), lambda i: (0, i, 0)),
            pl.BlockSpec((BM, 1), lambda i: (i, 0)),
            pl.BlockSpec((H, O), lambda i: (0, 0)),
            pl.BlockSpec((H, O), lambda i: (0, 0)),
            pl.BlockSpec((1, O), lambda i: (0, 0)),
            pl.BlockSpec((O, 2 * DEC_H), lambda i: (0, 0)),
            pl.BlockSpec((1, 2 * DEC_H), lambda i: (0, 0)),
        ],
        out_specs=[
            pl.BlockSpec((BM, DEC_H), lambda i: (i, 0)),
            pl.BlockSpec((BM, DEC_H), lambda i: (i, 0)),
        ],
        out_shape=[
            jax.ShapeDtypeStruct((N, DEC_H), jnp.float32),
            jax.ShapeDtypeStruct((N, DEC_H), jnp.float32),
        ],
    )(h0, h1, a0, a1, inv_cnt, w2lt, w2rt, b2l, dw1t, bpq)


LP = 102400
LW = LP // NW
DEC_CHUNK = 64
DEC_NCHUNK = LW // DEC_CHUNK


def _dec_body(p_hbm, q_hbm, sidx_hbm, didx_hbm, w_hbm, b2_hbm, out_hbm,
              sidx, didx, prows, qrows, wbuf, b2buf, tmp, outbuf, isem, gsem):
    cid = lax.axis_index("c")
    sid = lax.axis_index("s")
    wid = sid * NC + cid
    pltpu.sync_copy(w_hbm, wbuf)
    pltpu.sync_copy(b2_hbm, b2buf)
    wregs = [wbuf[pl.ds(k * 16, 16)] for k in range(DEC_H // 16)]
    b2v = b2buf[...]
    iota16x = lax.iota(jnp.int32, 16) * 16
    lbase = wid * LW

    def issue_idx(c, j):
        base = pl.multiple_of(lbase + c * DEC_CHUNK, 8)
        pltpu.async_copy(sidx_hbm.at[pl.ds(base, DEC_CHUNK)], sidx[j], isem[j])
        pltpu.async_copy(didx_hbm.at[pl.ds(base, DEC_CHUNK)], didx[j], isem[j])

    def wait_idx(j):
        pltpu.make_async_copy(sidx_hbm.at[pl.ds(0, DEC_CHUNK)], sidx[j], isem[j]).wait()
        pltpu.make_async_copy(didx_hbm.at[pl.ds(0, DEC_CHUNK)], didx[j], isem[j]).wait()

    def issue_gather(j):
        pltpu.async_copy(p_hbm.at[sidx[j]], prows[j], gsem[j])
        pltpu.async_copy(q_hbm.at[didx[j]], qrows[j], gsem[j])

    def wait_gather(j):
        pltpu.make_async_copy(p_hbm.at[sidx[j]], prows[j], gsem[j]).wait()
        pltpu.make_async_copy(q_hbm.at[didx[j]], qrows[j], gsem[j]).wait()

    def compute(c, j):
        pr = prows[j]
        qr = qrows[j]
        for g in range(DEC_CHUNK // 16):
            for e in range(16):
                row = g * 16 + e
                acc = None
                for k in range(DEC_H // 16):
                    pv = pr[row, pl.ds(k * 16, 16)]
                    qv = qr[row, pl.ds(k * 16, 16)]
                    t = jnp.maximum(pv + qv, 0.0) * wregs[k]
                    acc = t if acc is None else acc + t
                tmp[pl.ds(e * 16, 16)] = acc
            s = b2v
            for jj in range(16):
                s = s + plsc.load_gather(tmp, [iota16x + jj])
            outbuf[pl.ds(g * 16, 16)] = s
        base = pl.multiple_of(lbase + c * DEC_CHUNK, 8)
        pltpu.sync_copy(outbuf, out_hbm.at[pl.ds(base, DEC_CHUNK)])

    def triple(i, carry):
        for j3 in range(3):
            c = 3 * i + j3

            @pl.when(c <= DEC_NCHUNK - 1)
            def _(c=c, j=j3):
                issue_idx(c, j)

            @pl.when((c >= 1) & (c <= DEC_NCHUNK))
            def _(j=(j3 + 2) % 3):
                wait_idx(j)
                issue_gather(j)

            @pl.when((c >= 2) & (c <= DEC_NCHUNK + 1))
            def _(c=c, j=(j3 + 1) % 3):
                wait_gather(j)
                compute(c - 2, j)

        return carry

    lax.fori_loop(0, (DEC_NCHUNK + 4) // 3, triple, 0)


_decoder = pl.kernel(
    _dec_body,
    out_type=jax.ShapeDtypeStruct((LP,), jnp.float32),
    mesh=_MESH,
    compiler_params=_SC_PARAMS,
    scratch_types=[
        [pltpu.VMEM((DEC_CHUNK,), jnp.int32) for _ in range(3)],
        [pltpu.VMEM((DEC_CHUNK,), jnp.int32) for _ in range(3)],
        [pltpu.VMEM((DEC_CHUNK, DEC_H), jnp.float32) for _ in range(3)],
        [pltpu.VMEM((DEC_CHUNK, DEC_H), jnp.float32) for _ in range(3)],
        pltpu.VMEM((DEC_H,), jnp.float32),
        pltpu.VMEM((16,), jnp.float32),
        pltpu.VMEM((256,), jnp.float32),
        pltpu.VMEM((DEC_CHUNK,), jnp.float32),
        [pltpu.SemaphoreType.DMA for _ in range(3)],
        [pltpu.SemaphoreType.DMA for _ in range(3)],
    ],
)


def kernel(x, edge_index, edge_label_index, W1l, b1l, W1r, ln_g, ln_b,
           W2l, b2l, W2r, DW1, Db1, DW2, Db2):
    src = jnp.concatenate([edge_index[0], jnp.zeros((EP - E,), jnp.int32)])
    dst = jnp.concatenate([edge_index[1],
                           jnp.full((EP - E,), NP - 1, jnp.int32)])
    zeros2d = jnp.zeros((NP, D), jnp.float32)
    zeros1d = jnp.zeros((NP,), jnp.float32)
    ones = jnp.ones((SEG_CHUNK,), jnp.float32)

    parts1, cnt_parts = _seg_sum_counts(x, src, dst, zeros2d, zeros1d, ones)

    h0, h1, inv_cnt = _encoder1(
        x, parts1, cnt_parts.T, W1l.T, W1r.T,
        b1l.reshape(1, H), ln_g.reshape(1, H), ln_b.reshape(1, H))

    (a0,) = _seg_sum(h0, src, dst, zeros2d, zeros1d, ones)
    (a1,) = _seg_sum(h1, src, dst, zeros2d, zeros1d, ones)

    dw1t = jnp.concatenate([DW1[:, :O].T, DW1[:, O:].T], axis=1)
    bpq = jnp.concatenate([Db1, jnp.zeros((DEC_H,), jnp.float32)]).reshape(1, 2 * DEC_H)
    p_tab, q_tab = _encoder2(h0, h1, a0, a1, inv_cnt, W2l.T, W2r.T,
                             b2l.reshape(1, O), dw1t, bpq)

    s_idx = jnp.pad(edge_label_index[0], (0, LP - L))
    d_idx = jnp.pad(edge_label_index[1], (0, LP - L))
    w64 = DW2.reshape(DEC_H)
    b2_16 = jnp.broadcast_to(Db2, (16,))
    out = _decoder(p_tab, q_tab, s_idx, d_idx, w64, b2_16)
    return out[:L]


# speedup vs baseline: 1.3066x; 1.3066x over previous
import jax
import jax.numpy as jnp
from jax import lax
from jax.experimental import pallas as pl
from jax.experimental.pallas import tpu as pltpu
from jax.experimental.pallas import tpu_sc as plsc

N = 10000
E = 320000
L = 100000
D = 128
H = 256
O = 128
DEC_H = 64

NC = 2
NS = 16
NW = NC * NS

_MESH = plsc.VectorSubcoreMesh(core_axis_name="c", subcore_axis_name="s")
_SC_PARAMS = pltpu.CompilerParams(needs_layout_passes=False,
                                  use_tc_tiling_on_sc=False)

CW = 64
SEG_CHUNK = 120
NP = 10240
ROWS_PER_TILE = NP // NS
EPT = 20160
EP = NS * EPT
SEG_NCHUNK = EPT // SEG_CHUNK


def _make_seg_sum(with_counts: bool):
    out_type = [jax.ShapeDtypeStruct((NC, NP, CW), jnp.float32)]
    if with_counts:
        out_type.append(jax.ShapeDtypeStruct((NC, NP), jnp.float32))
    scratch = [
        pltpu.VMEM_SHARED((NP, CW), jnp.float32),
        pltpu.VMEM_SHARED((NP, CW), jnp.float32),
        [pltpu.VMEM((SEG_CHUNK,), jnp.int32) for _ in range(3)],
        [pltpu.VMEM((SEG_CHUNK,), jnp.int32) for _ in range(3)],
        [pltpu.VMEM((SEG_CHUNK, CW), jnp.float32) for _ in range(3)],
        [pltpu.SemaphoreType.DMA for _ in range(3)],
        [pltpu.SemaphoreType.DMA for _ in range(3)],
        [pltpu.SemaphoreType.DMA for _ in range(3)],
    ]
    if with_counts:
        scratch += [
            pltpu.VMEM_SHARED((NP,), jnp.float32),
            pltpu.VMEM((SEG_CHUNK,), jnp.float32),
        ]

    def body(tab0, tab1, src, dst, zeros2d, zeros1d, ones, *rest):
        if with_counts:
            (agg_out, cnt_out, tab_sp, agg_sp, sidx, didx, rows,
             isem, gsem, ssem, cnt_sp, ones_v) = rest
        else:
            (agg_out, tab_sp, agg_sp, sidx, didx, rows,
             isem, gsem, ssem) = rest
        cid = lax.axis_index("c")
        sid = lax.axis_index("s")
        rbase = pl.multiple_of(sid * ROWS_PER_TILE, 8)
        rslice = pl.ds(rbase, ROWS_PER_TILE)

        @pl.when(cid == 0)
        def _():
            pltpu.sync_copy(tab0.at[rslice], tab_sp.at[rslice])

        @pl.when(cid == 1)
        def _():
            pltpu.sync_copy(tab1.at[rslice], tab_sp.at[rslice])

        pltpu.sync_copy(zeros2d.at[rslice], agg_sp.at[rslice])
        if with_counts:
            pltpu.sync_copy(zeros1d.at[rslice], cnt_sp.at[rslice])
            pltpu.sync_copy(ones, ones_v)
        plsc.subcore_barrier()

        ebase = sid * EPT

        def issue_idx(c, j):
            base = pl.multiple_of(ebase + c * SEG_CHUNK, 8)
            pltpu.async_copy(src.at[pl.ds(base, SEG_CHUNK)], sidx[j], isem[j])
            pltpu.async_copy(dst.at[pl.ds(base, SEG_CHUNK)], didx[j], isem[j])

        def wait_idx(j):
            pltpu.make_async_copy(src.at[pl.ds(0, SEG_CHUNK)], sidx[j], isem[j]).wait()
            pltpu.make_async_copy(dst.at[pl.ds(0, SEG_CHUNK)], didx[j], isem[j]).wait()

        def issue_gather(j):
            pltpu.async_copy(tab_sp.at[sidx[j]], rows[j], gsem[j])

        def wait_gather(j):
            pltpu.make_async_copy(tab_sp.at[sidx[j]], rows[j], gsem[j]).wait()

        def issue_scatter(j):
            pltpu.async_copy(rows[j], agg_sp.at[didx[j]], ssem[j], add=True)
            if with_counts:
                pltpu.async_copy(ones_v, cnt_sp.at[didx[j]], ssem[j], add=True)

        def wait_scatter(j):
            pltpu.make_async_copy(rows[j], agg_sp.at[didx[j]], ssem[j]).wait()
            if with_counts:
                pltpu.make_async_copy(ones_v, cnt_sp.at[didx[j]], ssem[j]).wait()

        def triple(i, carry):
            for j3 in range(3):
                c = 3 * i + j3

                @pl.when(c >= 3)
                def _(j=j3):
                    wait_scatter(j)

                @pl.when(c <= SEG_NCHUNK - 1)
                def _(c=c, j=j3):
                    issue_idx(c, j)

                @pl.when((c >= 1) & (c <= SEG_NCHUNK))
                def _(j=(j3 + 2) % 3):
                    wait_idx(j)
                    issue_gather(j)

                @pl.when((c >= 2) & (c <= SEG_NCHUNK + 1))
                def _(j=(j3 + 1) % 3):
                    wait_gather(j)
                    issue_scatter(j)

            return carry

        lax.fori_loop(0, (SEG_NCHUNK + 3) // 3, triple, 0)

        plsc.subcore_barrier()
        pltpu.sync_copy(agg_sp.at[rslice], agg_out.at[cid, rslice])
        if with_counts:
            pltpu.sync_copy(cnt_sp.at[rslice], cnt_out.at[cid, rslice])

    return pl.kernel(body, out_type=tuple(out_type), mesh=_MESH,
                     compiler_params=_SC_PARAMS, scratch_types=scratch)


_seg_sum_counts = _make_seg_sum(True)
_seg_sum = _make_seg_sum(False)

BM = 1000


def _enc1_body(x_ref, a_ref, cnt_ref, w1lt_ref, w1rt_ref, b1l_ref, g_ref,
               b_ref, h0_ref, h1_ref, h2_ref, h3_ref, inv_ref):
    cnt = cnt_ref[:, 0]
    inv = 1.0 / jnp.maximum(cnt, 1.0)[:, None]
    w1lt = w1lt_ref[...]
    pre = (jnp.dot(a_ref[0] * inv, w1lt[:CW], preferred_element_type=jnp.float32)
           + jnp.dot(a_ref[1] * inv, w1lt[CW:], preferred_element_type=jnp.float32)
           + jnp.dot(x_ref[...], w1rt_ref[...], preferred_element_type=jnp.float32)
           + b1l_ref[...])
    mu = jnp.mean(pre, axis=-1, keepdims=True)
    var = jnp.mean((pre - mu) ** 2, axis=-1, keepdims=True)
    hh = (pre - mu) * lax.rsqrt(var + 1e-5) * g_ref[...] + b_ref[...]
    hh = jnp.maximum(hh, 0.0)
    h0_ref[...] = hh[:, :CW]
    h1_ref[...] = hh[:, CW:2 * CW]
    h2_ref[...] = hh[:, 2 * CW:3 * CW]
    h3_ref[...] = hh[:, 3 * CW:]
    inv_ref[...] = inv


def _encoder1(x, agg, cnt, w1lt, w1rt, b1l, ln_g, ln_b):
    grid = (N // BM,)
    return pl.pallas_call(
        _enc1_body,
        grid=grid,
        in_specs=[
            pl.BlockSpec((BM, D), lambda i: (i, 0)),
            pl.BlockSpec((NC, BM, CW), lambda i: (0, i, 0)),
            pl.BlockSpec((BM, NC), lambda i: (i, 0)),
            pl.BlockSpec((D, H), lambda i: (0, 0)),
            pl.BlockSpec((D, H), lambda i: (0, 0)),
            pl.BlockSpec((1, H), lambda i: (0, 0)),
            pl.BlockSpec((1, H), lambda i: (0, 0)),
            pl.BlockSpec((1, H), lambda i: (0, 0)),
        ],
        out_specs=[pl.BlockSpec((BM, CW), lambda i: (i, 0))] * 4
                  + [pl.BlockSpec((BM, 1), lambda i: (i, 0))],
        out_shape=[jax.ShapeDtypeStruct((N, CW), jnp.float32)] * 4
                  + [jax.ShapeDtypeStruct((N, 1), jnp.float32)],
    )(x, agg, cnt, w1lt, w1rt, b1l, ln_g, ln_b)


def _enc2_body(h0_ref, h1_ref, h2_ref, h3_ref, aA_ref, aB_ref, inv_ref,
               w2lt_ref, w2rt_ref, b2l_ref, dw1t_ref, bpq_ref, p_ref, q_ref):
    inv = inv_ref[...]
    w2lt = w2lt_ref[...]
    w2rt = w2rt_ref[...]
    hq = [h0_ref[...], h1_ref[...], h2_ref[...], h3_ref[...]]
    mq = [aA_ref[0] * inv, aA_ref[1] * inv, aB_ref[0] * inv, aB_ref[1] * inv]
    z = b2l_ref[...]
    for q in range(4):
        z = z + jnp.dot(mq[q], w2lt[q * CW:(q + 1) * CW],
                        preferred_element_type=jnp.float32)
        z = z + jnp.dot(hq[q], w2rt[q * CW:(q + 1) * CW],
                        preferred_element_type=jnp.float32)
    nrm = jnp.maximum(jnp.sqrt(jnp.sum(z * z, axis=-1, keepdims=True)), 1e-12)
    z = z / nrm
    pq = jnp.dot(z, dw1t_ref[...], preferred_element_type=jnp.float32) + bpq_ref[...]
    p_ref[...] = pq[:, :DEC_H]
    q_ref[...] = pq[:, DEC_H:]


def _encoder2(h, aggA, aggB, inv_cnt, w2lt, w2rt, b2l, dw1t, bpq):
    grid = (N // BM,)
    return pl.pallas_call(
        _enc2_body,
        grid=grid,
        in_specs=[pl.BlockSpec((BM, CW), lambda i: (i, 0))] * 4 + [
            pl.BlockSpec((NC, BM, CW), lambda i: (0, i, 0)),
            pl.BlockSpec((NC, BM, CW), lambda i: (0, i, 0)),
            pl.BlockSpec((BM, 1), lambda i: (i, 0)),
            pl.BlockSpec((H, O), lambda i: (0, 0)),
            pl.BlockSpec((H, O), lambda i: (0, 0)),
            pl.BlockSpec((1, O), lambda i: (0, 0)),
            pl.BlockSpec((O, 2 * DEC_H), lambda i: (0, 0)),
            pl.BlockSpec((1, 2 * DEC_H), lambda i: (0, 0)),
        ],
        out_specs=[
            pl.BlockSpec((BM, DEC_H), lambda i: (i, 0)),
            pl.BlockSpec((BM, DEC_H), lambda i: (i, 0)),
        ],
        out_shape=[
            jax.ShapeDtypeStruct((N, DEC_H), jnp.float32),
            jax.ShapeDtypeStruct((N, DEC_H), jnp.float32),
        ],
    )(*h, aggA, aggB, inv_cnt, w2lt, w2rt, b2l, dw1t, bpq)


LP = 102400
LW = LP // NW
DEC_CHUNK = 64
DEC_NCHUNK = LW // DEC_CHUNK


def _dec_body(p_hbm, q_hbm, sidx_hbm, didx_hbm, w_hbm, b2_hbm, out_hbm,
              p_sp, q_sp, sidx, didx, prows, qrows, wbuf, b2buf, tmp, outbuf,
              isem, gsem):
    cid = lax.axis_index("c")
    sid = lax.axis_index("s")
    wid = sid * NC + cid
    rbase = pl.multiple_of(sid * ROWS_PER_TILE, 8)
    rslice = pl.ds(rbase, ROWS_PER_TILE)
    pltpu.sync_copy(p_hbm.at[rslice], p_sp.at[rslice])
    pltpu.sync_copy(q_hbm.at[rslice], q_sp.at[rslice])
    pltpu.sync_copy(w_hbm, wbuf)
    pltpu.sync_copy(b2_hbm, b2buf)
    plsc.subcore_barrier()
    wregs = [wbuf[pl.ds(k * 16, 16)] for k in range(DEC_H // 16)]
    b2v = b2buf[...]
    iota16x = lax.iota(jnp.int32, 16) * 16
    lbase = wid * LW

    def issue_idx(c, j):
        base = pl.multiple_of(lbase + c * DEC_CHUNK, 8)
        pltpu.async_copy(sidx_hbm.at[pl.ds(base, DEC_CHUNK)], sidx[j], isem[j])
        pltpu.async_copy(didx_hbm.at[pl.ds(base, DEC_CHUNK)], didx[j], isem[j])

    def wait_idx(j):
        pltpu.make_async_copy(sidx_hbm.at[pl.ds(0, DEC_CHUNK)], sidx[j], isem[j]).wait()
        pltpu.make_async_copy(didx_hbm.at[pl.ds(0, DEC_CHUNK)], didx[j], isem[j]).wait()

    def issue_gather(j):
        pltpu.async_copy(p_sp.at[sidx[j]], prows[j], gsem[j])
        pltpu.async_copy(q_sp.at[didx[j]], qrows[j], gsem[j])

    def wait_gather(j):
        pltpu.make_async_copy(p_sp.at[sidx[j]], prows[j], gsem[j]).wait()
        pltpu.make_async_copy(q_sp.at[didx[j]], qrows[j], gsem[j]).wait()

    def compute(c, j):
        pr = prows[j]
        qr = qrows[j]
        for g in range(DEC_CHUNK // 16):
            for e in range(16):
                row = g * 16 + e
                acc = None
                for k in range(DEC_H // 16):
                    pv = pr[row, pl.ds(k * 16, 16)]
                    qv = qr[row, pl.ds(k * 16, 16)]
                    t = jnp.maximum(pv + qv, 0.0) * wregs[k]
                    acc = t if acc is None else acc + t
                tmp[pl.ds(e * 16, 16)] = acc
            s = b2v
            for jj in range(16):
                s = s + plsc.load_gather(tmp, [iota16x + jj])
            outbuf[pl.ds(g * 16, 16)] = s
        base = pl.multiple_of(lbase + c * DEC_CHUNK, 8)
        pltpu.sync_copy(outbuf, out_hbm.at[pl.ds(base, DEC_CHUNK)])

    def triple(i, carry):
        for j3 in range(3):
            c = 3 * i + j3

            @pl.when(c <= DEC_NCHUNK - 1)
            def _(c=c, j=j3):
                issue_idx(c, j)

            @pl.when((c >= 1) & (c <= DEC_NCHUNK))
            def _(j=(j3 + 2) % 3):
                wait_idx(j)
                issue_gather(j)

            @pl.when((c >= 2) & (c <= DEC_NCHUNK + 1))
            def _(c=c, j=(j3 + 1) % 3):
                wait_gather(j)
                compute(c - 2, j)

        return carry

    lax.fori_loop(0, (DEC_NCHUNK + 4) // 3, triple, 0)


_decoder = pl.kernel(
    _dec_body,
    out_type=jax.ShapeDtypeStruct((LP,), jnp.float32),
    mesh=_MESH,
    compiler_params=_SC_PARAMS,
    scratch_types=[
        pltpu.VMEM_SHARED((NP, DEC_H), jnp.float32),
        pltpu.VMEM_SHARED((NP, DEC_H), jnp.float32),
        [pltpu.VMEM((DEC_CHUNK,), jnp.int32) for _ in range(3)],
        [pltpu.VMEM((DEC_CHUNK,), jnp.int32) for _ in range(3)],
        [pltpu.VMEM((DEC_CHUNK, DEC_H), jnp.float32) for _ in range(3)],
        [pltpu.VMEM((DEC_CHUNK, DEC_H), jnp.float32) for _ in range(3)],
        pltpu.VMEM((DEC_H,), jnp.float32),
        pltpu.VMEM((16,), jnp.float32),
        pltpu.VMEM((256,), jnp.float32),
        pltpu.VMEM((DEC_CHUNK,), jnp.float32),
        [pltpu.SemaphoreType.DMA for _ in range(3)],
        [pltpu.SemaphoreType.DMA for _ in range(3)],
    ],
)


def _pad_rows(a):
    return jnp.pad(a, ((0, NP - N), (0, 0)))


def kernel(x, edge_index, edge_label_index, W1l, b1l, W1r, ln_g, ln_b,
           W2l, b2l, W2r, DW1, Db1, DW2, Db2):
    src = jnp.concatenate([edge_index[0], jnp.zeros((EP - E,), jnp.int32)])
    pad_dst = N + (jnp.arange(EP - E, dtype=jnp.int32) % (NP - N))
    dst = jnp.concatenate([edge_index[1], pad_dst])
    zeros2d = jnp.zeros((NP, CW), jnp.float32)
    zeros1d = jnp.zeros((NP,), jnp.float32)
    ones = jnp.ones((SEG_CHUNK,), jnp.float32)

    x0 = _pad_rows(x[:, :CW])
    x1 = _pad_rows(x[:, CW:])
    agg1, cnt = _seg_sum_counts(x0, x1, src, dst, zeros2d, zeros1d, ones)

    h0, h1, h2, h3, inv_cnt = _encoder1(
        x, agg1, cnt.T, W1l.T, W1r.T,
        b1l.reshape(1, H), ln_g.reshape(1, H), ln_b.reshape(1, H))

    (aggA,) = _seg_sum(_pad_rows(h0), _pad_rows(h1), src, dst,
                       zeros2d, zeros1d, ones)
    (aggB,) = _seg_sum(_pad_rows(h2), _pad_rows(h3), src, dst,
                       zeros2d, zeros1d, ones)

    dw1t = jnp.concatenate([DW1[:, :O].T, DW1[:, O:].T], axis=1)
    bpq = jnp.concatenate([Db1, jnp.zeros((DEC_H,), jnp.float32)]).reshape(1, 2 * DEC_H)
    p_tab, q_tab = _encoder2([h0, h1, h2, h3], aggA, aggB, inv_cnt,
                             W2l.T, W2r.T, b2l.reshape(1, O), dw1t, bpq)

    s_idx = jnp.pad(edge_label_index[0], (0, LP - L))
    d_idx = jnp.pad(edge_label_index[1], (0, LP - L))
    w64 = DW2.reshape(DEC_H)
    b2_16 = jnp.broadcast_to(Db2, (16,))
    out = _decoder(_pad_rows(p_tab), _pad_rows(q_tab), s_idx, d_idx,
                   w64, b2_16)
    return out[:L]


# speedup vs baseline: 1.6345x; 1.2509x over previous
import jax
import jax.numpy as jnp
import numpy as _np
from jax import lax
from jax.experimental import pallas as pl
from jax.experimental.pallas import tpu as pltpu
from jax.experimental.pallas import tpu_sc as plsc

N = 10000
E = 320000
L = 100000
D = 128
H = 256
O = 128
DEC_H = 64

NC = 2
NS = 16
NW = NC * NS

_MESH = plsc.VectorSubcoreMesh(core_axis_name="c", subcore_axis_name="s")
_SC_PARAMS = pltpu.CompilerParams(needs_layout_passes=False,
                                  use_tc_tiling_on_sc=False)

CW = 64
SEG_CHUNK = 80
NP = 10240
ROWS_PER_TILE = NP // NS
EPT = E // NS
SEG_NCHUNK = EPT // SEG_CHUNK


def _make_seg_sum(with_counts: bool, table_width: int, col_base: int):
    out_type = [jax.ShapeDtypeStruct((NC, NP, CW), jnp.float32)]
    if with_counts:
        out_type.append(jax.ShapeDtypeStruct((NC, NS, NP), jnp.float32))
    scratch = [
        pltpu.VMEM_SHARED((NP, CW), jnp.float32),
        pltpu.VMEM_SHARED((NP, CW), jnp.float32),
        [pltpu.VMEM((SEG_CHUNK,), jnp.int32) for _ in range(4)],
        [pltpu.VMEM((SEG_CHUNK,), jnp.int32) for _ in range(4)],
        [pltpu.VMEM((SEG_CHUNK, CW), jnp.float32) for _ in range(4)],
        [pltpu.SemaphoreType.DMA for _ in range(4)],
        [pltpu.SemaphoreType.DMA for _ in range(4)],
        [pltpu.SemaphoreType.DMA for _ in range(4)],
    ]
    if with_counts:
        scratch += [
            pltpu.VMEM((NP,), jnp.float32),
        ]

    def body(table, ei, zeros2d, zeros1d, *rest):
        if with_counts:
            (agg_out, cnt_out, tab_sp, agg_sp, sidx, didx, rows,
             isem, gsem, ssem, cnt_loc) = rest
        else:
            (agg_out, tab_sp, agg_sp, sidx, didx, rows,
             isem, gsem, ssem) = rest
        cid = lax.axis_index("c")
        sid = lax.axis_index("s")
        rbase = pl.multiple_of(sid * ROWS_PER_TILE, 8)
        rslice = pl.ds(rbase, ROWS_PER_TILE)

        col0 = col_base + cid * CW
        pltpu.sync_copy(table.at[rslice, pl.ds(col0, CW)], tab_sp.at[rslice])

        pltpu.sync_copy(zeros2d.at[rslice], agg_sp.at[rslice])
        if with_counts:
            pltpu.sync_copy(zeros1d, cnt_loc)
        plsc.subcore_barrier()

        ebase = sid * EPT

        def issue_idx(c, j):
            base = pl.multiple_of(ebase + c * SEG_CHUNK, 8)
            pltpu.async_copy(ei.at[0, pl.ds(base, SEG_CHUNK)], sidx[j], isem[j])
            pltpu.async_copy(ei.at[1, pl.ds(base, SEG_CHUNK)], didx[j], isem[j])

        def wait_idx(j):
            pltpu.make_async_copy(ei.at[0, pl.ds(0, SEG_CHUNK)], sidx[j], isem[j]).wait()
            pltpu.make_async_copy(ei.at[1, pl.ds(0, SEG_CHUNK)], didx[j], isem[j]).wait()

        def issue_gather(j):
            pltpu.async_copy(tab_sp.at[sidx[j]], rows[j], gsem[j])

        def wait_gather(j):
            pltpu.make_async_copy(tab_sp.at[sidx[j]], rows[j], gsem[j]).wait()

        ones16 = jnp.ones((16,), jnp.float32)
        tail_mask = lax.iota(jnp.int32, 16) >= (16 - SEG_CHUNK % 16)

        def issue_scatter(j):
            pltpu.async_copy(rows[j], agg_sp.at[didx[j]], ssem[j], add=True)
            if with_counts:
                for q in range(SEG_CHUNK // 16):
                    idxv = didx[j][pl.ds(q * 16, 16)]
                    plsc.addupdate_scatter(cnt_loc, [idxv], ones16)
                if SEG_CHUNK % 16:
                    idxv = didx[j][pl.ds(SEG_CHUNK - 16, 16)]
                    plsc.addupdate_scatter(cnt_loc, [idxv], ones16,
                                           mask=tail_mask)

        def wait_scatter(j):
            pltpu.make_async_copy(rows[j], agg_sp.at[didx[j]], ssem[j]).wait()

        def quad(i, carry):
            for j4 in range(4):
                c = 4 * i + j4

                @pl.when((c >= 4) & (c <= SEG_NCHUNK + 3))
                def _(j=j4):
                    wait_scatter(j)

                @pl.when(c <= SEG_NCHUNK - 1)
                def _(c=c, j=j4):
                    issue_idx(c, j)

                @pl.when((c >= 1) & (c <= SEG_NCHUNK))
                def _(j=(j4 + 3) % 4):
                    wait_idx(j)
                    issue_gather(j)

                @pl.when((c >= 2) & (c <= SEG_NCHUNK + 1))
                def _(j=(j4 + 2) % 4):
                    wait_gather(j)
                    issue_scatter(j)

            return carry

        lax.fori_loop(0, (SEG_NCHUNK + 7) // 4, quad, 0)

        plsc.subcore_barrier()
        pltpu.sync_copy(agg_sp.at[rslice], agg_out.at[cid, rslice])
        if with_counts:
            pltpu.sync_copy(cnt_loc, cnt_out.at[cid, sid])

    return pl.kernel(body, out_type=tuple(out_type), mesh=_MESH,
                     compiler_params=_SC_PARAMS, scratch_types=scratch)


_seg_sum_counts = _make_seg_sum(True, D, 0)
_seg_sum_lo = _make_seg_sum(False, H, 0)
_seg_sum_hi = _make_seg_sum(False, H, 2 * CW)

BM = 1000


def _enc1_body(x_ref, a_ref, cnt_ref, w1lt_ref, w1rt_ref, b1l_ref, g_ref,
               b_ref, h_ref, inv_ref):
    cnt = jnp.sum(cnt_ref[...], axis=1, keepdims=True)
    inv = 1.0 / jnp.maximum(cnt, 1.0)
    w1lt = w1lt_ref[...]
    pre = (jnp.dot(a_ref[0] * inv, w1lt[:CW], preferred_element_type=jnp.float32)
           + jnp.dot(a_ref[1] * inv, w1lt[CW:], preferred_element_type=jnp.float32)
           + jnp.dot(x_ref[...], w1rt_ref[...], preferred_element_type=jnp.float32)
           + b1l_ref[...])
    mu = jnp.mean(pre, axis=-1, keepdims=True)
    var = jnp.mean((pre - mu) ** 2, axis=-1, keepdims=True)
    hh = (pre - mu) * lax.rsqrt(var + 1e-5) * g_ref[...] + b_ref[...]
    hh = jnp.maximum(hh, 0.0)
    h_ref[...] = hh
    inv_ref[...] = inv


def _encoder1(x, agg, cnt, w1lt, w1rt, b1l, ln_g, ln_b):
    grid = (N // BM,)
    return pl.pallas_call(
        _enc1_body,
        grid=grid,
        in_specs=[
            pl.BlockSpec((BM, D), lambda i: (i, 0)),
            pl.BlockSpec((NC, BM, CW), lambda i: (0, i, 0)),
            pl.BlockSpec((BM, NS), lambda i: (i, 0)),
            pl.BlockSpec((D, H), lambda i: (0, 0)),
            pl.BlockSpec((D, H), lambda i: (0, 0)),
            pl.BlockSpec((1, H), lambda i: (0, 0)),
            pl.BlockSpec((1, H), lambda i: (0, 0)),
            pl.BlockSpec((1, H), lambda i: (0, 0)),
        ],
        out_specs=[
            pl.BlockSpec((BM, H), lambda i: (i, 0)),
            pl.BlockSpec((BM, 1), lambda i: (i, 0)),
        ],
        out_shape=[
            jax.ShapeDtypeStruct((NP, H), jnp.float32),
            jax.ShapeDtypeStruct((NP, 1), jnp.float32),
        ],
    )(x, agg, cnt, w1lt, w1rt, b1l, ln_g, ln_b)


def _enc2_body(h_ref, aA_ref, aB_ref, inv_ref,
               w2lt_ref, w2rt_ref, b2l_ref, dw1t_ref, bpq_ref, p_ref, q_ref):
    inv = inv_ref[...]
    w2lt = w2lt_ref[...]
    w2rt = w2rt_ref[...]
    mq = [aA_ref[0] * inv, aA_ref[1] * inv, aB_ref[0] * inv, aB_ref[1] * inv]
    z = (jnp.dot(h_ref[...], w2rt, preferred_element_type=jnp.float32)
         + b2l_ref[...])
    for q in range(4):
        z = z + jnp.dot(mq[q], w2lt[q * CW:(q + 1) * CW],
                        preferred_element_type=jnp.float32)
    nrm = jnp.maximum(jnp.sqrt(jnp.sum(z * z, axis=-1, keepdims=True)), 1e-12)
    z = z / nrm
    pq = jnp.dot(z, dw1t_ref[...], preferred_element_type=jnp.float32) + bpq_ref[...]
    p_ref[...] = pq[:, :DEC_H]
    q_ref[...] = pq[:, DEC_H:]


def _encoder2(h, aggA, aggB, inv_cnt, w2lt, w2rt, b2l, dw1t, bpq):
    grid = (N // BM,)
    return pl.pallas_call(
        _enc2_body,
        grid=grid,
        in_specs=[pl.BlockSpec((BM, H), lambda i: (i, 0))] + [
            pl.BlockSpec((NC, BM, CW), lambda i: (0, i, 0)),
            pl.BlockSpec((NC, BM, CW), lambda i: (0, i, 0)),
            pl.BlockSpec((BM, 1), lambda i: (i, 0)),
            pl.BlockSpec((H, O), lambda i: (0, 0)),
            pl.BlockSpec((H, O), lambda i: (0, 0)),
            pl.BlockSpec((1, O), lambda i: (0, 0)),
            pl.BlockSpec((O, 2 * DEC_H), lambda i: (0, 0)),
            pl.BlockSpec((1, 2 * DEC_H), lambda i: (0, 0)),
        ],
        out_specs=[
            pl.BlockSpec((BM, DEC_H), lambda i: (i, 0)),
            pl.BlockSpec((BM, DEC_H), lambda i: (i, 0)),
        ],
        out_shape=[
            jax.ShapeDtypeStruct((NP, DEC_H), jnp.float32),
            jax.ShapeDtypeStruct((NP, DEC_H), jnp.float32),
        ],
    )(h, aggA, aggB, inv_cnt, w2lt, w2rt, b2l, dw1t, bpq)


LP = 102400
LW = LP // NW
DEC_CHUNK = 64
DEC_NCHUNK = LW // DEC_CHUNK


def _dec_body(p_hbm, q_hbm, sidx_hbm, didx_hbm, w_hbm, b2_hbm, out_hbm,
              p_sp, q_sp, sidx, didx, prows, qrows, wbuf, b2buf, tmp, outbuf,
              isem, gsem):
    cid = lax.axis_index("c")
    sid = lax.axis_index("s")
    wid = sid * NC + cid
    rbase = pl.multiple_of(sid * ROWS_PER_TILE, 8)
    rslice = pl.ds(rbase, ROWS_PER_TILE)
    pltpu.sync_copy(p_hbm.at[rslice], p_sp.at[rslice])
    pltpu.sync_copy(q_hbm.at[rslice], q_sp.at[rslice])
    pltpu.sync_copy(w_hbm, wbuf)
    pltpu.sync_copy(b2_hbm, b2buf)
    plsc.subcore_barrier()
    wregs = [wbuf[pl.ds(k * 16, 16)] for k in range(DEC_H // 16)]
    b2v = b2buf[...]
    iota16x = lax.iota(jnp.int32, 16) * 16
    lbase = wid * LW

    def issue_idx(c, j):
        base = pl.multiple_of(lbase + c * DEC_CHUNK, 8)
        pltpu.async_copy(sidx_hbm.at[pl.ds(base, DEC_CHUNK)], sidx[j], isem[j])
        pltpu.async_copy(didx_hbm.at[pl.ds(base, DEC_CHUNK)], didx[j], isem[j])

    def wait_idx(j):
        pltpu.make_async_copy(sidx_hbm.at[pl.ds(0, DEC_CHUNK)], sidx[j], isem[j]).wait()
        pltpu.make_async_copy(didx_hbm.at[pl.ds(0, DEC_CHUNK)], didx[j], isem[j]).wait()

    def issue_gather(j):
        pltpu.async_copy(p_sp.at[sidx[j]], prows[j], gsem[j])
        pltpu.async_copy(q_sp.at[didx[j]], qrows[j], gsem[j])

    def wait_gather(j):
        pltpu.make_async_copy(p_sp.at[sidx[j]], prows[j], gsem[j]).wait()
        pltpu.make_async_copy(q_sp.at[didx[j]], qrows[j], gsem[j]).wait()

    def compute(c, j):
        pr = prows[j]
        qr = qrows[j]
        for g in range(DEC_CHUNK // 16):
            for e in range(16):
                row = g * 16 + e
                acc = None
                for k in range(DEC_H // 16):
                    pv = pr[row, pl.ds(k * 16, 16)]
                    qv = qr[row, pl.ds(k * 16, 16)]
                    t = jnp.maximum(pv + qv, 0.0) * wregs[k]
                    acc = t if acc is None else acc + t
                tmp[pl.ds(e * 16, 16)] = acc
            s = b2v
            for jj in range(16):
                s = s + plsc.load_gather(tmp, [iota16x + jj])
            outbuf[pl.ds(g * 16, 16)] = s
        base = pl.multiple_of(lbase + c * DEC_CHUNK, 8)
        pltpu.sync_copy(outbuf, out_hbm.at[pl.ds(base, DEC_CHUNK)])

    def quad(i, carry):
        for j4 in range(4):
            c = 4 * i + j4

            @pl.when(c <= DEC_NCHUNK - 1)
            def _(c=c, j=j4):
                issue_idx(c, j)

            @pl.when((c >= 1) & (c <= DEC_NCHUNK))
            def _(j=(j4 + 3) % 4):
                wait_idx(j)
                issue_gather(j)

            @pl.when((c >= 2) & (c <= DEC_NCHUNK + 1))
            def _(c=c, j=(j4 + 2) % 4):
                wait_gather(j)
                compute(c - 2, j)

        return carry

    lax.fori_loop(0, (DEC_NCHUNK + 5) // 4, quad, 0)


_decoder = pl.kernel(
    _dec_body,
    out_type=jax.ShapeDtypeStruct((LP,), jnp.float32),
    mesh=_MESH,
    compiler_params=_SC_PARAMS,
    scratch_types=[
        pltpu.VMEM_SHARED((NP, DEC_H), jnp.float32),
        pltpu.VMEM_SHARED((NP, DEC_H), jnp.float32),
        [pltpu.VMEM((DEC_CHUNK,), jnp.int32) for _ in range(4)],
        [pltpu.VMEM((DEC_CHUNK,), jnp.int32) for _ in range(4)],
        [pltpu.VMEM((DEC_CHUNK, DEC_H), jnp.float32) for _ in range(4)],
        [pltpu.VMEM((DEC_CHUNK, DEC_H), jnp.float32) for _ in range(4)],
        pltpu.VMEM((DEC_H,), jnp.float32),
        pltpu.VMEM((16,), jnp.float32),
        pltpu.VMEM((256,), jnp.float32),
        pltpu.VMEM((DEC_CHUNK,), jnp.float32),
        [pltpu.SemaphoreType.DMA for _ in range(4)],
        [pltpu.SemaphoreType.DMA for _ in range(4)],
    ],
)


def kernel(x, edge_index, edge_label_index, W1l, b1l, W1r, ln_g, ln_b,
           W2l, b2l, W2r, DW1, Db1, DW2, Db2):
    zeros2d = jnp.zeros((NP, CW), jnp.float32)
    zeros1d = jnp.zeros((NP,), jnp.float32)

    xp = jnp.pad(x, ((0, NP - N), (0, 0)))
    agg1, cnt = _seg_sum_counts(xp, edge_index, zeros2d, zeros1d)

    h, inv_cnt = _encoder1(
        xp, agg1, cnt[0].T, W1l.T, W1r.T,
        b1l.reshape(1, H), ln_g.reshape(1, H), ln_b.reshape(1, H))

    (aggA,) = _seg_sum_lo(h, edge_index, zeros2d, zeros1d)
    (aggB,) = _seg_sum_hi(h, edge_index, zeros2d, zeros1d)

    dw1t = jnp.concatenate([DW1[:, :O].T, DW1[:, O:].T], axis=1)
    bpq = jnp.concatenate([Db1, jnp.zeros((DEC_H,), jnp.float32)]).reshape(1, 2 * DEC_H)
    p_tab, q_tab = _encoder2(h, aggA, aggB, inv_cnt,
                             W2l.T, W2r.T, b2l.reshape(1, O), dw1t, bpq)

    s_idx = jnp.pad(edge_label_index[0], (0, LP - L))
    d_idx = jnp.pad(edge_label_index[1], (0, LP - L))
    w64 = DW2.reshape(DEC_H)
    b2_16 = jnp.broadcast_to(Db2, (16,))
    out = _decoder(p_tab, q_tab, s_idx, d_idx, w64, b2_16)
    return out[:L]
